# Initial kernel scaffold; baseline (speedup 1.0000x reference)
import jax
import jax.numpy as jnp
from jax.experimental import pallas as pl


def kernel(x, gate_w, w1, b1, wo, bo):
    raise NotImplementedError("write your pallas kernel here")


# speedup vs baseline: 1.9448x; 1.9448x over previous
import functools

import jax
import jax.numpy as jnp
from jax.experimental import pallas as pl
from jax.experimental.pallas import tpu as pltpu

DIM = 1024
HID = 2048
NE = 8
S = 2048

TC_COL = 512
TR_OUT = 512


def _gate_body(x_ref, gw_ref, m_ref, sw_ref):
    x = x_ref[...]
    logits = jnp.dot(x, gw_ref[...], preferred_element_type=jnp.float32)
    mx = jnp.max(logits, axis=1, keepdims=True)
    ex = jnp.exp(logits - mx)
    s = ex / jnp.sum(ex, axis=1, keepdims=True)

    r = jax.lax.broadcasted_iota(jnp.int32, (NE, NE), 0)
    c = jax.lax.broadcasted_iota(jnp.int32, (NE, NE), 1)
    lte = (r <= c).astype(jnp.float32)

    m1 = jnp.max(s, axis=1, keepdims=True)
    t1 = (s == m1).astype(jnp.float32)
    c1 = jnp.dot(t1, lte, preferred_element_type=jnp.float32)
    h1 = t1 * (c1 == 1.0).astype(jnp.float32)

    s2 = jnp.where(h1 > 0, -1.0, s)
    m2 = jnp.max(s2, axis=1, keepdims=True)
    t2 = (s2 == m2).astype(jnp.float32)
    c2 = jnp.dot(t2, lte, preferred_element_type=jnp.float32)
    h2 = t2 * (c2 == 1.0).astype(jnp.float32)

    denom = m1 + m2 + 1e-9
    v1 = m1 / denom
    v2 = m2 / denom
    m_ref[...] = h1 * v1 + h2 * v2
    sw_ref[...] = v1 + v2


def _expert_body(m_ref, x_ref, wa_ref, wb_ref, ba_ref, bb_ref, acc_ref):
    e = pl.program_id(1)
    x = x_ref[...]
    ha = jnp.dot(x, wa_ref[0], preferred_element_type=jnp.float32) + ba_ref[0]
    hb = jnp.dot(x, wb_ref[0], preferred_element_type=jnp.float32) + bb_ref[0]
    glu = ha * jax.nn.sigmoid(hb)

    lane = jax.lax.broadcasted_iota(jnp.int32, (1, NE), 1)
    sel = (lane == e).astype(jnp.float32)
    w_col = jnp.sum(m_ref[...] * sel, axis=1, keepdims=True)
    contrib = glu * w_col

    @pl.when(e == 0)
    def _():
        acc_ref[...] = contrib

    @pl.when(e > 0)
    def _():
        acc_ref[...] += contrib


def _proj_body(acc_ref, wo_ref, sw_ref, bo_ref, out_ref):
    out_ref[...] = (
        jnp.dot(acc_ref[...], wo_ref[...], preferred_element_type=jnp.float32)
        + sw_ref[...] * bo_ref[...]
    )


@jax.jit
def kernel(x, gate_w, w1, b1, wo, bo):
    B = x.shape[0]
    x2 = x.reshape(S, DIM)

    m, sw = pl.pallas_call(
        _gate_body,
        out_shape=(
            jax.ShapeDtypeStruct((S, NE), jnp.float32),
            jax.ShapeDtypeStruct((S, 1), jnp.float32),
        ),
    )(x2, gate_w)

    n_col = HID // TC_COL
    b1_3 = b1.reshape(NE, 1, 2 * HID)
    acc = pl.pallas_call(
        _expert_body,
        grid=(n_col, NE),
        in_specs=[
            pl.BlockSpec((S, NE), lambda c, e: (0, 0)),
            pl.BlockSpec((S, DIM), lambda c, e: (0, 0)),
            pl.BlockSpec((1, DIM, TC_COL), lambda c, e: (e, 0, c)),
            pl.BlockSpec((1, DIM, TC_COL), lambda c, e: (e, 0, c + n_col)),
            pl.BlockSpec((1, 1, TC_COL), lambda c, e: (e, 0, c)),
            pl.BlockSpec((1, 1, TC_COL), lambda c, e: (e, 0, c + n_col)),
        ],
        out_specs=pl.BlockSpec((S, TC_COL), lambda c, e: (0, c)),
        out_shape=jax.ShapeDtypeStruct((S, HID), jnp.float32),
    )(m, x2, w1, w1, b1_3, b1_3)

    out = pl.pallas_call(
        _proj_body,
        grid=(S // TR_OUT,),
        in_specs=[
            pl.BlockSpec((TR_OUT, HID), lambda r: (r, 0)),
            pl.BlockSpec((HID, DIM), lambda r: (0, 0)),
            pl.BlockSpec((TR_OUT, 1), lambda r: (r, 0)),
            pl.BlockSpec((1, DIM), lambda r: (0, 0)),
        ],
        out_specs=pl.BlockSpec((TR_OUT, DIM), lambda r: (r, 0)),
        out_shape=jax.ShapeDtypeStruct((S, DIM), jnp.float32),
    )(acc, wo, sw, bo.reshape(1, DIM))

    return out.reshape(B, S, DIM)
